# Initial kernel scaffold; baseline (speedup 1.0000x reference)
import functools

import jax
import jax.numpy as jnp
from jax import lax
from jax.experimental import pallas as pl
from jax.experimental.pallas import tpu as pltpu
from jax.experimental.pallas import tpu_sc as plsc

_N = 10000
_E = 320000
_D = 128
_HID = 128
_OUT = 16
_B = 10
_M = 1000
_C = 64

_NC = 2
_NS = 16
_NW = _NC * _NS
_EPW = _E // _NW
_CH = 80
_NCHUNK = _EPW // _CH
_RPS = _N // _NS
_ZR = 25


def _sc_edge_agg(dst_r, src_r, x):
    mesh = plsc.VectorSubcoreMesh(core_axis_name="c", subcore_axis_name="s")

    @functools.partial(
        pl.kernel,
        out_type=jax.ShapeDtypeStruct((_NC, _N, _D), jnp.float32),
        mesh=mesh,
        scratch_types=[
            pltpu.VMEM((_NCHUNK, _CH), jnp.int32),
            pltpu.VMEM((_NCHUNK, _CH), jnp.int32),
            pltpu.VMEM((_CH, _D), jnp.float32),
            pltpu.VMEM((_ZR, _D), jnp.float32),
            pltpu.VMEM_SHARED((_N, _D), jnp.float32),
            pltpu.SemaphoreType.DMA,
        ],
    )
    def k(dst_hbm, src_hbm, x_hbm, out_hbm, src_v, dst_v, rows_v, zero_v,
          accum, sem):
        cid = lax.axis_index("c")
        sid = lax.axis_index("s")
        wid = sid * _NC + cid

        for i in range(_ZR):
            for j in range(_D // 16):
                zero_v[i, j * 16:(j + 1) * 16] = jnp.zeros((16,), jnp.float32)

        def zbody(t, carry):
            pltpu.sync_copy(zero_v, accum.at[pl.ds(sid * _RPS + t * _ZR, _ZR)])
            return carry

        lax.fori_loop(0, _RPS // _ZR, zbody, 0)

        pltpu.sync_copy(src_hbm.at[wid], src_v)
        pltpu.sync_copy(dst_hbm.at[wid], dst_v)

        plsc.subcore_barrier()

        def body(j, carry):
            pltpu.async_copy(x_hbm.at[src_v.at[j]], rows_v, sem).wait()
            pltpu.sync_copy(rows_v, accum.at[dst_v.at[j]], add=True)
            return carry

        lax.fori_loop(0, _NCHUNK, body, 0)

        plsc.subcore_barrier()

        pltpu.sync_copy(accum.at[pl.ds(sid * _RPS, _RPS)],
                        out_hbm.at[cid, pl.ds(sid * _RPS, _RPS)])

    return k(dst_r, src_r, x)


def _tc_body(part_ref, x_ref, gid_ref, cl_ref, hm_ref, idx_ref, eps_ref,
             ws_ref, w1_ref, b1_ref, g1_ref, be1_ref, w2_ref, b2_ref,
             ge_ref, bee_ref, wp_ref, bp_ref, h_out, ph_out, sc_out):
    f32 = jnp.float32

    g_oh = (lax.broadcasted_iota(jnp.int32, (_N, 16), 1)
            == gid_ref[...]).astype(f32)
    i_oh = (lax.broadcasted_iota(jnp.int32, (16, _M), 1)
            == idx_ref[...]).astype(f32)

    cl = cl_ref[...]
    cl_idx = jnp.dot(i_oh, cl, preferred_element_type=f32)
    t1 = lax.dot_general(cl_idx, cl, (((1,), (1,)), ((), ())),
                         preferred_element_type=f32)
    t2 = jnp.dot(t1, hm_ref[...], preferred_element_type=f32)

    p = part_ref[0] + part_ref[1]
    p = (1.0 + ws_ref[0]) * p + (1.0 + eps_ref[0]) * x_ref[...]
    p = p + (1.0 + ws_ref[1]) * jnp.dot(g_oh, t2, preferred_element_type=f32)

    h1 = jnp.dot(p, w1_ref[...], preferred_element_type=f32) + b1_ref[...]
    mu1 = jnp.mean(h1, axis=0, keepdims=True)
    v1 = jnp.mean((h1 - mu1) * (h1 - mu1), axis=0, keepdims=True)
    hh = jnp.maximum(
        (h1 - mu1) * lax.rsqrt(v1 + 1e-5) * g1_ref[...] + be1_ref[...], 0.0)

    pr = jnp.dot(hh, w2_ref[...], preferred_element_type=f32) + b2_ref[...]
    mu2 = jnp.mean(pr, axis=0, keepdims=True)
    v2 = jnp.mean((pr - mu2) * (pr - mu2), axis=0, keepdims=True)
    h = jnp.maximum(
        (pr - mu2) * lax.rsqrt(v2 + 1e-5) * ge_ref[...] + bee_ref[...], 0.0)

    ph = lax.dot_general(g_oh, h, (((0,), (0,)), ((), ())),
                         preferred_element_type=f32)
    sc = jnp.dot(ph, wp_ref[...], preferred_element_type=f32) + bp_ref[...]

    h_out[...] = h
    ph_out[...] = ph
    sc_out[...] = sc


def kernel(x, edge_index, graph_ids, Cl, H, idx, eps, ws, W1, b1, g1, be1,
           W2, b2, g_e, be_e, Wp, bp):
    dst_r = edge_index[0].reshape(_NW, _NCHUNK, _CH)
    src_r = edge_index[1].reshape(_NW, _NCHUNK, _CH)

    part = _sc_edge_agg(dst_r, src_r, x)

    gid2d = graph_ids.reshape(_N, 1)
    idx16 = jnp.zeros((16, 1), jnp.int32).at[:_B, 0].set(idx)

    vm = pl.BlockSpec(memory_space=pltpu.VMEM)
    sm = pl.BlockSpec(memory_space=pltpu.SMEM)

    h, ph, sc = pl.pallas_call(
        _tc_body,
        out_shape=[
            jax.ShapeDtypeStruct((_N, _D), jnp.float32),
            jax.ShapeDtypeStruct((16, _D), jnp.float32),
            jax.ShapeDtypeStruct((16, _OUT), jnp.float32),
        ],
        in_specs=[vm, vm, vm, vm, vm, vm, sm, sm,
                  vm, vm, vm, vm, vm, vm, vm, vm, vm, vm],
        out_specs=[vm, vm, vm],
    )(part, x, gid2d, Cl, H, idx16, eps, ws,
      W1, b1.reshape(1, _HID), g1.reshape(1, _HID), be1.reshape(1, _HID),
      W2, b2.reshape(1, _D), g_e.reshape(1, _D), be_e.reshape(1, _D),
      Wp, bp.reshape(1, _OUT))

    return (sc[:_B], ph[:_B], h)


# speedup vs baseline: 6.2114x; 6.2114x over previous
import functools

import jax
import jax.numpy as jnp
from jax import lax
from jax.experimental import pallas as pl
from jax.experimental.pallas import tpu as pltpu
from jax.experimental.pallas import tpu_sc as plsc

_N = 10000
_E = 320000
_D = 128
_HID = 128
_OUT = 16
_B = 10
_M = 1000
_C = 64

_NC = 2
_NS = 16
_NW = _NC * _NS
_EPW = _E // _NW
_CH = 80
_NCHUNK = _EPW // _CH
_NP = 10240
_RPS = _NP // _NS
_ZR = 40


def _sc_edge_agg(dst_r, src_r, x):
    mesh = plsc.VectorSubcoreMesh(core_axis_name="c", subcore_axis_name="s")

    @functools.partial(
        pl.kernel,
        out_type=jax.ShapeDtypeStruct((_NC, _NP, _D), jnp.float32),
        mesh=mesh,
        scratch_types=[
            pltpu.VMEM((_NCHUNK, _CH), jnp.int32),
            pltpu.VMEM((_NCHUNK, _CH), jnp.int32),
            pltpu.VMEM((_CH, _D), jnp.float32),
            pltpu.VMEM((_ZR, _D), jnp.float32),
            pltpu.VMEM_SHARED((_NP, _D), jnp.float32),
            pltpu.SemaphoreType.DMA,
        ],
    )
    def k(dst_hbm, src_hbm, x_hbm, out_hbm, src_v, dst_v, rows_v, zero_v,
          accum, sem):
        cid = lax.axis_index("c")
        sid = lax.axis_index("s")
        wid = sid * _NC + cid

        for i in range(_ZR):
            for j in range(_D // 16):
                zero_v[i, j * 16:(j + 1) * 16] = jnp.zeros((16,), jnp.float32)

        def zbody(t, carry):
            pltpu.sync_copy(zero_v, accum.at[pl.ds(sid * _RPS + t * _ZR, _ZR)])
            return carry

        lax.fori_loop(0, _RPS // _ZR, zbody, 0)

        pltpu.sync_copy(src_hbm.at[wid], src_v)
        pltpu.sync_copy(dst_hbm.at[wid], dst_v)

        plsc.subcore_barrier()

        def body(j, carry):
            pltpu.async_copy(x_hbm.at[src_v.at[j]], rows_v, sem).wait()
            pltpu.sync_copy(rows_v, accum.at[dst_v.at[j]], add=True)
            return carry

        lax.fori_loop(0, _NCHUNK, body, 0)

        plsc.subcore_barrier()

        pltpu.sync_copy(accum.at[pl.ds(sid * _RPS, _RPS)],
                        out_hbm.at[cid, pl.ds(sid * _RPS, _RPS)])

    return k(dst_r, src_r, x)


def _tc_body(part_ref, x_ref, gid_ref, cl_ref, hm_ref, idx_ref, eps_ref,
             ws_ref, w1_ref, b1_ref, g1_ref, be1_ref, w2_ref, b2_ref,
             ge_ref, bee_ref, wp_ref, bp_ref, h_out, ph_out, sc_out):
    f32 = jnp.float32

    g_oh = (lax.broadcasted_iota(jnp.int32, (_N, 16), 1)
            == gid_ref[...]).astype(f32)
    i_oh = (lax.broadcasted_iota(jnp.int32, (16, _M), 1)
            == idx_ref[...]).astype(f32)

    cl = cl_ref[...]
    cl_idx = jnp.dot(i_oh, cl, preferred_element_type=f32, precision=lax.Precision.HIGHEST)
    t1 = lax.dot_general(cl_idx, cl, (((1,), (1,)), ((), ())),
                         preferred_element_type=f32, precision=lax.Precision.HIGHEST)
    t2 = jnp.dot(t1, hm_ref[...], preferred_element_type=f32, precision=lax.Precision.HIGHEST)

    p = part_ref[0, 0:_N, :] + part_ref[1, 0:_N, :]
    p = (1.0 + ws_ref[0]) * p + (1.0 + eps_ref[0]) * x_ref[...]
    p = p + (1.0 + ws_ref[1]) * jnp.dot(g_oh, t2, preferred_element_type=f32, precision=lax.Precision.HIGHEST)

    h1 = jnp.dot(p, w1_ref[...], preferred_element_type=f32, precision=lax.Precision.HIGHEST) + b1_ref[...]
    mu1 = jnp.mean(h1, axis=0, keepdims=True)
    v1 = jnp.mean((h1 - mu1) * (h1 - mu1), axis=0, keepdims=True)
    hh = jnp.maximum(
        (h1 - mu1) * lax.rsqrt(v1 + 1e-5) * g1_ref[...] + be1_ref[...], 0.0)

    pr = jnp.dot(hh, w2_ref[...], preferred_element_type=f32, precision=lax.Precision.HIGHEST) + b2_ref[...]
    mu2 = jnp.mean(pr, axis=0, keepdims=True)
    v2 = jnp.mean((pr - mu2) * (pr - mu2), axis=0, keepdims=True)
    h = jnp.maximum(
        (pr - mu2) * lax.rsqrt(v2 + 1e-5) * ge_ref[...] + bee_ref[...], 0.0)

    ph = lax.dot_general(g_oh, h, (((0,), (0,)), ((), ())),
                         preferred_element_type=f32, precision=lax.Precision.HIGHEST)
    sc = jnp.dot(ph, wp_ref[...], preferred_element_type=f32, precision=lax.Precision.HIGHEST) + bp_ref[...]

    h_out[...] = h
    ph_out[...] = ph
    sc_out[...] = sc


def kernel(x, edge_index, graph_ids, Cl, H, idx, eps, ws, W1, b1, g1, be1,
           W2, b2, g_e, be_e, Wp, bp):
    dst_r = edge_index[0].reshape(_NW, _NCHUNK, _CH)
    src_r = edge_index[1].reshape(_NW, _NCHUNK, _CH)

    part = _sc_edge_agg(dst_r, src_r, x)

    gid2d = graph_ids.reshape(_N, 1)
    idx16 = jnp.zeros((16, 1), jnp.int32).at[:_B, 0].set(idx)

    vm = pl.BlockSpec(memory_space=pltpu.VMEM)
    sm = pl.BlockSpec(memory_space=pltpu.SMEM)

    h, ph, sc = pl.pallas_call(
        _tc_body,
        out_shape=[
            jax.ShapeDtypeStruct((_N, _D), jnp.float32),
            jax.ShapeDtypeStruct((16, _D), jnp.float32),
            jax.ShapeDtypeStruct((16, _OUT), jnp.float32),
        ],
        in_specs=[vm, vm, vm, vm, vm, vm, sm, sm,
                  vm, vm, vm, vm, vm, vm, vm, vm, vm, vm],
        out_specs=[vm, vm, vm],
    )(part, x, gid2d, Cl, H, idx16, eps, ws,
      W1, b1.reshape(1, _HID), g1.reshape(1, _HID), be1.reshape(1, _HID),
      W2, b2.reshape(1, _D), g_e.reshape(1, _D), be_e.reshape(1, _D),
      Wp, bp.reshape(1, _OUT))

    return (sc[:_B], ph[:_B], h)
